# Initial kernel scaffold; baseline (speedup 1.0000x reference)
import functools

import jax
import jax.numpy as jnp
from jax import lax
from jax.experimental import pallas as pl
from jax.experimental.pallas import tpu as pltpu
from jax.experimental.pallas import tpu_sc as plsc

N_NODES = 10000
N_EDGES = 320000
IN_CH = 128
HID_CH = 128
OUT_CH = 64
N_GRAPHS = 64

NC = 2
NS = 16
NW = NC * NS

K_EDGE = 128
NCHUNK = 80
E_PER_TILE = K_EDGE * NCHUNK
E_PAD = NW * E_PER_TILE

ACC_ROWS = 10240
ZROWS_PER_TILE = ACC_ROWS // NS
ZB = ZROWS_PER_TILE // 2
OUT_ROWS_PER_TILE = N_NODES // NS

CNT_ROWS = 10016


def _sc_mesh():
    return plsc.VectorSubcoreMesh(core_axis_name="c", subcore_axis_name="s")


def _make_deg_kernel():
    @functools.partial(
        pl.kernel,
        out_type=jax.ShapeDtypeStruct((NW, N_NODES), jnp.float32),
        mesh=_sc_mesh(),
        scratch_types=[
            pltpu.VMEM((E_PER_TILE,), jnp.int32),
            pltpu.VMEM((CNT_ROWS,), jnp.float32),
        ],
    )
    def deg_kernel(dst_hbm, out_hbm, idx_v, cnt_v):
        c = lax.axis_index("c")
        s = lax.axis_index("s")
        wid = c * NS + s
        pltpu.sync_copy(dst_hbm.at[pl.ds(wid * E_PER_TILE, E_PER_TILE)], idx_v)

        zeros16 = jnp.zeros((16,), jnp.float32)
        ones16 = jnp.full((16,), 1.0, jnp.float32)

        def zero_body(i, _):
            cnt_v[pl.ds(i * 16, 16)] = zeros16
            return 0

        lax.fori_loop(0, CNT_ROWS // 16, zero_body, 0)

        def scat_body(i, _):
            idx = idx_v[pl.ds(i * 16, 16)]
            plsc.addupdate_scatter(cnt_v, [idx], ones16)
            return 0

        lax.fori_loop(0, E_PER_TILE // 16, scat_body, 0)
        pltpu.sync_copy(cnt_v.at[pl.ds(0, N_NODES)], out_hbm.at[wid])

    return deg_kernel


def _make_scatter_kernel(width):
    @functools.partial(
        pl.kernel,
        out_type=jax.ShapeDtypeStruct((NC, N_NODES, width), jnp.float32),
        mesh=_sc_mesh(),
        scratch_types=[
            pltpu.VMEM((NCHUNK, K_EDGE), jnp.int32),
            pltpu.VMEM((NCHUNK, K_EDGE), jnp.int32),
            pltpu.VMEM((2, K_EDGE, width), jnp.float32),
            pltpu.VMEM((ZB, width), jnp.float32),
            pltpu.VMEM_SHARED((ACC_ROWS, width), jnp.float32),
            pltpu.SemaphoreType.DMA,
            pltpu.SemaphoreType.DMA,
        ],
    )
    def scatter_kernel(y_hbm, src_hbm, dst_hbm, zeros_hbm, out_hbm,
                       src_v, dst_v, rows_v, z_v, acc_sh, sem0, sem1):
        c = lax.axis_index("c")
        s = lax.axis_index("s")
        wid = c * NS + s
        sems = (sem0, sem1)

        pltpu.sync_copy(src_hbm.at[wid], src_v)
        pltpu.sync_copy(dst_hbm.at[wid], dst_v)

        pltpu.sync_copy(zeros_hbm, z_v)
        zbase = s * ZROWS_PER_TILE
        pltpu.sync_copy(z_v, acc_sh.at[pl.ds(zbase, ZB)])
        pltpu.sync_copy(z_v, acc_sh.at[pl.ds(zbase + ZB, ZB)])
        plsc.subcore_barrier()

        for b in range(2):
            pltpu.async_copy(y_hbm.at[src_v.at[b]], rows_v.at[b], sems[b])

        def body(jj, _):
            for b in range(2):
                j = jj * 2 + b
                pltpu.make_async_copy(
                    y_hbm.at[src_v.at[j]], rows_v.at[b], sems[b]).wait()
                pltpu.sync_copy(rows_v.at[b], acc_sh.at[dst_v.at[j]], add=True)

                @pl.when(j + 2 < NCHUNK)
                def _():
                    pltpu.async_copy(
                        y_hbm.at[src_v.at[j + 2]], rows_v.at[b], sems[b])
            return 0

        lax.fori_loop(0, NCHUNK // 2, body, 0)
        plsc.subcore_barrier()

        obase = s * OUT_ROWS_PER_TILE
        pltpu.sync_copy(acc_sh.at[pl.ds(obase, OUT_ROWS_PER_TILE)],
                        out_hbm.at[c, pl.ds(obase, OUT_ROWS_PER_TILE)])

    return scatter_kernel


_deg_kernel = _make_deg_kernel()
_scatter_l1 = _make_scatter_kernel(HID_CH)
_scatter_l2 = _make_scatter_kernel(OUT_CH)


def _tc1_body(x_ref, w1_ref, parts_ref, y1_ref, dinv_ref):
    ones = jnp.ones((NW, 1), jnp.float32)
    deg = lax.dot_general(parts_ref[...], ones,
                          (((0,), (0,)), ((), ())),
                          precision=lax.Precision.HIGHEST,
                          preferred_element_type=jnp.float32)
    dinv = lax.rsqrt(deg)
    xw = jnp.dot(x_ref[...], w1_ref[...],
                 precision=lax.Precision.HIGHEST,
                 preferred_element_type=jnp.float32)
    y1_ref[...] = xw * dinv
    dinv_ref[...] = dinv


def _tc2_body(p_ref, y1_ref, dinv_ref, b1_ref, w2_ref, y2_ref):
    dinv = dinv_ref[...]
    srow = p_ref[0] + p_ref[1] + y1_ref[...]
    h = jnp.maximum(srow * dinv + b1_ref[...], 0.0)
    y2_ref[...] = jnp.dot(h, w2_ref[...],
                          precision=lax.Precision.HIGHEST,
                          preferred_element_type=jnp.float32) * dinv


def _tc3_body(p_ref, y2_ref, dinv_ref, b2_ref, batch_ref, out_ref):
    srow = p_ref[0] + p_ref[1] + y2_ref[...]
    h = srow * dinv_ref[...] + b2_ref[...]
    onehot = (batch_ref[...] == lax.broadcasted_iota(
        jnp.int32, (N_NODES, N_GRAPHS), 1)).astype(jnp.float32)
    seg = lax.dot_general(onehot, h, (((0,), (0,)), ((), ())),
                          precision=lax.Precision.HIGHEST,
                          preferred_element_type=jnp.float32)
    counts = lax.dot_general(onehot, jnp.ones((N_NODES, 1), jnp.float32),
                             (((0,), (0,)), ((), ())),
                             precision=lax.Precision.HIGHEST,
                             preferred_element_type=jnp.float32)
    out_ref[...] = seg / jnp.maximum(counts, 1.0)


_tc1 = pl.pallas_call(
    _tc1_body,
    out_shape=(jax.ShapeDtypeStruct((N_NODES, HID_CH), jnp.float32),
               jax.ShapeDtypeStruct((N_NODES, 1), jnp.float32)))

_tc2 = pl.pallas_call(
    _tc2_body,
    out_shape=jax.ShapeDtypeStruct((N_NODES, OUT_CH), jnp.float32))

_tc3 = pl.pallas_call(
    _tc3_body,
    out_shape=jax.ShapeDtypeStruct((N_GRAPHS, OUT_CH), jnp.float32))


def kernel(x, edge_index, batch, W1, b1, W2, b2):
    src = edge_index[0].astype(jnp.int32)
    dst = edge_index[1].astype(jnp.int32)
    pad = E_PAD - N_EDGES
    src_p = jnp.concatenate([src, jnp.zeros((pad,), jnp.int32)])
    dst_p = jnp.concatenate([dst, jnp.full((pad,), N_NODES, jnp.int32)])
    src3 = src_p.reshape(NW, NCHUNK, K_EDGE)
    dst3 = dst_p.reshape(NW, NCHUNK, K_EDGE)

    zeros_hid = jnp.zeros((ZB, HID_CH), jnp.float32)
    zeros_out = jnp.zeros((ZB, OUT_CH), jnp.float32)

    deg_parts = _deg_kernel(dst_p)
    y1, dinv = _tc1(x, W1, deg_parts)
    p1 = _scatter_l1(y1, src3, dst3, zeros_hid)
    y2 = _tc2(p1, y1, dinv, b1.reshape(1, HID_CH), W2)
    p2 = _scatter_l2(y2, src3, dst3, zeros_out)
    return _tc3(p2, y2, dinv, b2.reshape(1, OUT_CH),
                batch.astype(jnp.int32).reshape(N_NODES, 1))


# speedup vs baseline: 8.9998x; 8.9998x over previous
import functools

import jax
import jax.numpy as jnp
from jax import lax
from jax.experimental import pallas as pl
from jax.experimental.pallas import tpu as pltpu
from jax.experimental.pallas import tpu_sc as plsc

N_NODES = 10000
N_EDGES = 320000
IN_CH = 128
HID_CH = 128
OUT_CH = 64
N_GRAPHS = 64

NC = 2
NS = 16
NW = NC * NS

K_EDGE = 128
NCHUNK = 80
E_PER_TILE = K_EDGE * NCHUNK
E_PAD = NW * E_PER_TILE

ACC_ROWS = 10112
ZROWS_PER_TILE = ACC_ROWS // NS
HALF = NCHUNK // 2

CNT_ROWS = 10240


def _sc_mesh():
    return plsc.VectorSubcoreMesh(core_axis_name="c", subcore_axis_name="s",
                                  num_cores=NC, num_subcores=NS)


def _make_deg_kernel():
    @functools.partial(
        pl.kernel,
        out_type=jax.ShapeDtypeStruct((NW, CNT_ROWS), jnp.float32),
        mesh=_sc_mesh(),
        scratch_types=[
            pltpu.VMEM((E_PER_TILE,), jnp.int32),
            pltpu.VMEM((CNT_ROWS,), jnp.float32),
        ],
        compiler_params=pltpu.CompilerParams(needs_layout_passes=False),
    )
    def deg_kernel(dst_hbm, out_hbm, idx_v, cnt_v):
        c = lax.axis_index("c")
        s = lax.axis_index("s")
        wid = c * NS + s
        pltpu.sync_copy(dst_hbm.at[pl.ds(wid * E_PER_TILE, E_PER_TILE)], idx_v)

        zeros16 = jnp.zeros((16,), jnp.float32)
        ones16 = jnp.full((16,), 1.0, jnp.float32)

        def zero_body(i, _):
            cnt_v[pl.ds(i * 16, 16)] = zeros16
            return 0

        lax.fori_loop(0, CNT_ROWS // 16, zero_body, 0)

        def scat_body(i, _):
            idx = idx_v[pl.ds(i * 16, 16)]
            plsc.addupdate_scatter(cnt_v, [idx], ones16)
            return 0

        lax.fori_loop(0, E_PER_TILE // 16, scat_body, 0)
        pltpu.sync_copy(cnt_v, out_hbm.at[wid])

    return deg_kernel


def _make_scatter_kernel(width):
    @functools.partial(
        pl.kernel,
        out_type=jax.ShapeDtypeStruct((NC, ACC_ROWS, width), jnp.float32),
        mesh=_sc_mesh(),
        scratch_types=[
            pltpu.VMEM((HALF, K_EDGE), jnp.int32),
            pltpu.VMEM((HALF, K_EDGE), jnp.int32),
            pltpu.VMEM((2, K_EDGE, width), jnp.float32),
            pltpu.VMEM_SHARED((ACC_ROWS, width), jnp.float32),
            pltpu.SemaphoreType.DMA,
            pltpu.SemaphoreType.DMA,
        ],
    )
    def scatter_kernel(y_hbm, src_hbm, dst_hbm, zeros_hbm, out_hbm,
                       src_v, dst_v, rows_v, acc_sh, sem0, sem1):
        c = lax.axis_index("c")
        s = lax.axis_index("s")
        wid = c * NS + s
        sems = (sem0, sem1)

        zbase = s * ZROWS_PER_TILE
        pltpu.sync_copy(zeros_hbm, acc_sh.at[pl.ds(zbase, ZROWS_PER_TILE)])
        plsc.subcore_barrier()

        for half in range(2):
            pltpu.sync_copy(src_hbm.at[wid, pl.ds(half * HALF, HALF)], src_v)
            pltpu.sync_copy(dst_hbm.at[wid, pl.ds(half * HALF, HALF)], dst_v)

            for b in range(2):
                pltpu.async_copy(y_hbm.at[src_v.at[b]], rows_v.at[b], sems[b])

            def body(jj, _):
                for b in range(2):
                    j = jj * 2 + b
                    pltpu.make_async_copy(
                        y_hbm.at[src_v.at[j]], rows_v.at[b], sems[b]).wait()
                    pltpu.sync_copy(rows_v.at[b], acc_sh.at[dst_v.at[j]],
                                    add=True)

                    @pl.when(j + 2 < HALF)
                    def _():
                        pltpu.async_copy(
                            y_hbm.at[src_v.at[j + 2]], rows_v.at[b], sems[b])
                return 0

            lax.fori_loop(0, HALF // 2, body, 0)

        plsc.subcore_barrier()
        pltpu.sync_copy(acc_sh.at[pl.ds(zbase, ZROWS_PER_TILE)],
                        out_hbm.at[c, pl.ds(zbase, ZROWS_PER_TILE)])

    return scatter_kernel


_make_deg_kernel = functools.cache(_make_deg_kernel)
_make_scatter_kernel = functools.cache(_make_scatter_kernel)


def _tc1_body(x_ref, w1_ref, parts_ref, y1_ref, dinv_ref):
    ones = jnp.ones((NW, 1), jnp.float32)
    deg = lax.dot_general(parts_ref[...], ones,
                          (((0,), (0,)), ((), ())),
                          precision=lax.Precision.HIGHEST,
                          preferred_element_type=jnp.float32)
    deg = lax.slice(deg, (0, 0), (N_NODES, 1)) + 1.0
    dinv = lax.rsqrt(deg)
    xw = jnp.dot(x_ref[...], w1_ref[...],
                 precision=lax.Precision.HIGHEST,
                 preferred_element_type=jnp.float32)
    y1_ref[...] = xw * dinv
    dinv_ref[...] = dinv


def _tc2_body(p_ref, y1_ref, dinv_ref, b1_ref, w2_ref, y2_ref):
    dinv = dinv_ref[...]
    psum = p_ref[0] + p_ref[1]
    srow = lax.slice(psum, (0, 0), (N_NODES, HID_CH)) + y1_ref[...]
    h = jnp.maximum(srow * dinv + b1_ref[...], 0.0)
    y2_ref[...] = jnp.dot(h, w2_ref[...],
                          precision=lax.Precision.HIGHEST,
                          preferred_element_type=jnp.float32) * dinv


def _tc3_body(p_ref, y2_ref, dinv_ref, b2_ref, batch_ref, out_ref):
    psum = p_ref[0] + p_ref[1]
    srow = (lax.slice(psum, (0, 0), (N_NODES, OUT_CH))
            + lax.slice(y2_ref[...], (0, 0), (N_NODES, OUT_CH)))
    h = srow * dinv_ref[...] + b2_ref[...]
    onehot = (batch_ref[...] == lax.broadcasted_iota(
        jnp.int32, (N_NODES, N_GRAPHS), 1)).astype(jnp.float32)
    seg = lax.dot_general(onehot, h, (((0,), (0,)), ((), ())),
                          precision=lax.Precision.HIGHEST,
                          preferred_element_type=jnp.float32)
    counts = lax.dot_general(onehot, jnp.ones((N_NODES, 1), jnp.float32),
                             (((0,), (0,)), ((), ())),
                             precision=lax.Precision.HIGHEST,
                             preferred_element_type=jnp.float32)
    out_ref[...] = seg / jnp.maximum(counts, 1.0)


_tc1 = pl.pallas_call(
    _tc1_body,
    out_shape=(jax.ShapeDtypeStruct((N_NODES, HID_CH), jnp.float32),
               jax.ShapeDtypeStruct((N_NODES, 1), jnp.float32)))

_tc2 = pl.pallas_call(
    _tc2_body,
    out_shape=jax.ShapeDtypeStruct((N_NODES, HID_CH), jnp.float32))

_tc3 = pl.pallas_call(
    _tc3_body,
    out_shape=jax.ShapeDtypeStruct((N_GRAPHS, OUT_CH), jnp.float32))


def kernel(x, edge_index, batch, W1, b1, W2, b2):
    src = edge_index[0].astype(jnp.int32)
    dst = edge_index[1].astype(jnp.int32)
    pad = E_PAD - N_EDGES
    src_p = jnp.concatenate([src, jnp.zeros((pad,), jnp.int32)])
    dst_p = jnp.concatenate([dst, jnp.full((pad,), N_NODES, jnp.int32)])
    src3 = src_p.reshape(NW, NCHUNK, K_EDGE)
    dst3 = dst_p.reshape(NW, NCHUNK, K_EDGE)

    zeros_hid = jnp.zeros((ZROWS_PER_TILE, HID_CH), jnp.float32)
    w2p = jnp.pad(W2, ((0, 0), (0, HID_CH - OUT_CH)))

    deg_kernel = _make_deg_kernel()
    scatter = _make_scatter_kernel(HID_CH)

    deg_parts = deg_kernel(dst_p)
    y1, dinv = _tc1(x, W1, deg_parts)
    p1 = scatter(y1, src3, dst3, zeros_hid)
    y2 = _tc2(p1, y1, dinv, b1.reshape(1, HID_CH), w2p)
    p2 = scatter(y2, src3, dst3, zeros_hid)
    return _tc3(p2, y2, dinv, b2.reshape(1, OUT_CH),
                batch.astype(jnp.int32).reshape(N_NODES, 1))


# speedup vs baseline: 9.0600x; 1.0067x over previous
import functools

import jax
import jax.numpy as jnp
from jax import lax
from jax.experimental import pallas as pl
from jax.experimental.pallas import tpu as pltpu
from jax.experimental.pallas import tpu_sc as plsc

N_NODES = 10000
N_EDGES = 320000
IN_CH = 128
HID_CH = 128
OUT_CH = 64
N_GRAPHS = 64

NC = 2
NS = 16
NW = NC * NS

K_EDGE = 128
NCHUNK = 80
E_PER_TILE = K_EDGE * NCHUNK
E_PAD = NW * E_PER_TILE

ACC_ROWS = 10112
ZROWS_PER_TILE = ACC_ROWS // NS
HALF = NCHUNK // 2

CNT_ROWS = 10240


def _sc_mesh():
    return plsc.VectorSubcoreMesh(core_axis_name="c", subcore_axis_name="s",
                                  num_cores=NC, num_subcores=NS)


def _make_deg_kernel():
    @functools.partial(
        pl.kernel,
        out_type=jax.ShapeDtypeStruct((NW, CNT_ROWS), jnp.float32),
        mesh=_sc_mesh(),
        scratch_types=[
            pltpu.VMEM((E_PER_TILE,), jnp.int32),
            pltpu.VMEM((CNT_ROWS,), jnp.float32),
        ],
        compiler_params=pltpu.CompilerParams(needs_layout_passes=False),
    )
    def deg_kernel(dst_hbm, out_hbm, idx_v, cnt_v):
        c = lax.axis_index("c")
        s = lax.axis_index("s")
        wid = c * NS + s
        pltpu.sync_copy(dst_hbm.at[pl.ds(wid * E_PER_TILE, E_PER_TILE)], idx_v)

        zeros16 = jnp.zeros((16,), jnp.float32)
        ones16 = jnp.full((16,), 1.0, jnp.float32)

        def zero_body(i, _):
            cnt_v[pl.ds(i * 16, 16)] = zeros16
            return 0

        lax.fori_loop(0, CNT_ROWS // 16, zero_body, 0)

        def scat_body(i, _):
            idx = idx_v[pl.ds(i * 16, 16)]
            plsc.addupdate_scatter(cnt_v, [idx], ones16)
            return 0

        lax.fori_loop(0, E_PER_TILE // 16, scat_body, 0)
        pltpu.sync_copy(cnt_v, out_hbm.at[wid])

    return deg_kernel


def _make_scatter_kernel(width):
    @functools.partial(
        pl.kernel,
        out_type=jax.ShapeDtypeStruct((NC, ACC_ROWS, width), jnp.float32),
        mesh=_sc_mesh(),
        scratch_types=[
            pltpu.VMEM((HALF, K_EDGE), jnp.int32),
            pltpu.VMEM((HALF, K_EDGE), jnp.int32),
            pltpu.VMEM((2, K_EDGE, width), jnp.float32),
            pltpu.VMEM_SHARED((ACC_ROWS, width), jnp.float32),
            pltpu.SemaphoreType.DMA,
            pltpu.SemaphoreType.DMA,
        ],
    )
    def scatter_kernel(y_hbm, src_hbm, dst_hbm, zeros_hbm, out_hbm,
                       src_v, dst_v, rows_v, acc_sh, sem0, sem1):
        c = lax.axis_index("c")
        s = lax.axis_index("s")
        wid = c * NS + s
        sems = (sem0, sem1)

        zbase = s * ZROWS_PER_TILE
        pltpu.sync_copy(zeros_hbm, acc_sh.at[pl.ds(zbase, ZROWS_PER_TILE)])
        plsc.subcore_barrier()

        for half in range(2):
            pltpu.sync_copy(src_hbm.at[wid, pl.ds(half * HALF, HALF)], src_v)
            pltpu.sync_copy(dst_hbm.at[wid, pl.ds(half * HALF, HALF)], dst_v)

            for b in range(2):
                pltpu.async_copy(y_hbm.at[src_v.at[b]], rows_v.at[b], sems[b])

            def body(jj, _):
                for b in range(2):
                    j = jj * 2 + b
                    pltpu.make_async_copy(
                        y_hbm.at[src_v.at[j]], rows_v.at[b], sems[b]).wait()
                    pltpu.sync_copy(rows_v.at[b], acc_sh.at[dst_v.at[j]],
                                    add=True)

                    @pl.when(j + 2 < HALF)
                    def _():
                        pltpu.async_copy(
                            y_hbm.at[src_v.at[j + 2]], rows_v.at[b], sems[b])
                return 0

            lax.fori_loop(0, HALF // 2, body, 0)

        plsc.subcore_barrier()
        pltpu.sync_copy(acc_sh.at[pl.ds(zbase, ZROWS_PER_TILE)],
                        out_hbm.at[c, pl.ds(zbase, ZROWS_PER_TILE)])

    return scatter_kernel


_make_deg_kernel = functools.cache(_make_deg_kernel)
_make_scatter_kernel = functools.cache(_make_scatter_kernel)


def _tc1_body(x_ref, w1_ref, parts_ref, y1_ref, dinv_ref):
    ones = jnp.ones((NW, 1), jnp.float32)
    deg = lax.dot_general(parts_ref[...], ones,
                          (((0,), (0,)), ((), ())),
                          precision=lax.Precision.HIGHEST,
                          preferred_element_type=jnp.float32)
    deg = lax.slice(deg, (0, 0), (N_NODES, 1)) + 1.0
    dinv = lax.rsqrt(deg)
    xw = jnp.dot(x_ref[...], w1_ref[...],
                 precision=lax.Precision.HIGHEST,
                 preferred_element_type=jnp.float32)
    y1_ref[...] = xw * dinv
    dinv_ref[...] = dinv


def _tc2_body(p_ref, y1_ref, dinv_ref, b1_ref, w2_ref, y2_ref):
    dinv = dinv_ref[...]
    psum = p_ref[0] + p_ref[1]
    srow = lax.slice(psum, (0, 0), (N_NODES, HID_CH)) + y1_ref[...]
    h = jnp.maximum(srow * dinv + b1_ref[...], 0.0)
    y2_ref[...] = jnp.dot(h, w2_ref[...],
                          precision=lax.Precision.HIGHEST,
                          preferred_element_type=jnp.float32) * dinv


def _tc3_body(p_ref, y2_ref, dinv_ref, b2_ref, batch_ref, out_ref):
    psum = p_ref[0] + p_ref[1]
    srow = (lax.slice(psum, (0, 0), (N_NODES, OUT_CH))
            + lax.slice(y2_ref[...], (0, 0), (N_NODES, OUT_CH)))
    h = srow * dinv_ref[...] + b2_ref[...]
    onehot = (batch_ref[...] == lax.broadcasted_iota(
        jnp.int32, (N_NODES, N_GRAPHS), 1)).astype(jnp.float32)
    seg = lax.dot_general(onehot, h, (((0,), (0,)), ((), ())),
                          precision=lax.Precision.HIGHEST,
                          preferred_element_type=jnp.float32)
    counts = lax.dot_general(onehot, jnp.ones((N_NODES, 1), jnp.float32),
                             (((0,), (0,)), ((), ())),
                             precision=lax.Precision.HIGHEST,
                             preferred_element_type=jnp.float32)
    out_ref[...] = seg / jnp.maximum(counts, 1.0)


_tc1 = pl.pallas_call(
    _tc1_body,
    out_shape=(jax.ShapeDtypeStruct((N_NODES, HID_CH), jnp.float32),
               jax.ShapeDtypeStruct((N_NODES, 1), jnp.float32)))

_tc2 = pl.pallas_call(
    _tc2_body,
    out_shape=jax.ShapeDtypeStruct((N_NODES, HID_CH), jnp.float32))

_tc3 = pl.pallas_call(
    _tc3_body,
    out_shape=jax.ShapeDtypeStruct((N_GRAPHS, OUT_CH), jnp.float32))


def kernel(x, edge_index, batch, W1, b1, W2, b2):
    src = edge_index[0].astype(jnp.int32)
    dst = edge_index[1].astype(jnp.int32)
    pad = E_PAD - N_EDGES
    src_p = jnp.concatenate([src, jnp.zeros((pad,), jnp.int32)])
    pad_dst = N_NODES + jnp.arange(pad, dtype=jnp.int32) % (ACC_ROWS - N_NODES)
    dst_p = jnp.concatenate([dst, pad_dst])
    src3 = src_p.reshape(NW, NCHUNK, K_EDGE)
    dst3 = dst_p.reshape(NW, NCHUNK, K_EDGE)

    zeros_hid = jnp.zeros((ZROWS_PER_TILE, HID_CH), jnp.float32)
    w2p = jnp.pad(W2, ((0, 0), (0, HID_CH - OUT_CH)))

    deg_kernel = _make_deg_kernel()
    scatter = _make_scatter_kernel(HID_CH)

    deg_parts = deg_kernel(dst_p)
    y1, dinv = _tc1(x, W1, deg_parts)
    p1 = scatter(y1, src3, dst3, zeros_hid)
    y2 = _tc2(p1, y1, dinv, b1.reshape(1, HID_CH), w2p)
    p2 = scatter(y2, src3, dst3, zeros_hid)
    return _tc3(p2, y2, dinv, b2.reshape(1, OUT_CH),
                batch.astype(jnp.int32).reshape(N_NODES, 1))


# speedup vs baseline: 12.2248x; 1.3493x over previous
import functools

import jax
import jax.numpy as jnp
from jax import lax
from jax.experimental import pallas as pl
from jax.experimental.pallas import tpu as pltpu
from jax.experimental.pallas import tpu_sc as plsc

N_NODES = 10000
N_EDGES = 320000
IN_CH = 128
HID_CH = 128
OUT_CH = 64
N_GRAPHS = 64

NC = 2
NS = 16
NW = NC * NS

K_EDGE = 128
NCHUNK = 80
E_PER_TILE = K_EDGE * NCHUNK
E_PAD = NW * E_PER_TILE

ACC_ROWS = 10112
ZROWS_PER_TILE = ACC_ROWS // NS
HALF = NCHUNK // 2

CNT_ROWS = 10240


def _sc_mesh():
    return plsc.VectorSubcoreMesh(core_axis_name="c", subcore_axis_name="s",
                                  num_cores=NC, num_subcores=NS)


def _make_deg_kernel():
    @functools.partial(
        pl.kernel,
        out_type=jax.ShapeDtypeStruct((NW, CNT_ROWS), jnp.float32),
        mesh=_sc_mesh(),
        scratch_types=[
            pltpu.VMEM((E_PER_TILE,), jnp.int32),
            pltpu.VMEM((CNT_ROWS,), jnp.float32),
        ],
        compiler_params=pltpu.CompilerParams(needs_layout_passes=False),
    )
    def deg_kernel(dst_hbm, out_hbm, idx_v, cnt_v):
        c = lax.axis_index("c")
        s = lax.axis_index("s")
        wid = c * NS + s
        pltpu.sync_copy(dst_hbm.at[pl.ds(wid * E_PER_TILE, E_PER_TILE)], idx_v)

        zeros16 = jnp.zeros((16,), jnp.float32)
        ones16 = jnp.full((16,), 1.0, jnp.float32)

        def zero_body(i, _):
            cnt_v[pl.ds(i * 16, 16)] = zeros16
            return 0

        lax.fori_loop(0, CNT_ROWS // 16, zero_body, 0)

        def scat_body(i, _):
            idx = idx_v[pl.ds(i * 16, 16)]
            plsc.addupdate_scatter(cnt_v, [idx], ones16)
            return 0

        lax.fori_loop(0, E_PER_TILE // 16, scat_body, 0)
        pltpu.sync_copy(cnt_v, out_hbm.at[wid])

    return deg_kernel


def _make_scatter_kernel(width):
    @functools.partial(
        pl.kernel,
        out_type=jax.ShapeDtypeStruct((NC, ACC_ROWS, width), jnp.float32),
        mesh=_sc_mesh(),
        scratch_types=[
            pltpu.VMEM((HALF, K_EDGE), jnp.int32),
            pltpu.VMEM((HALF, K_EDGE), jnp.int32),
            pltpu.VMEM((2, K_EDGE, width), jnp.float32),
            pltpu.VMEM_SHARED((ACC_ROWS, width), jnp.float32),
            pltpu.SemaphoreType.DMA,
            pltpu.SemaphoreType.DMA,
        ],
        compiler_params=(None if width == 128 else
                         pltpu.CompilerParams(use_tc_tiling_on_sc=False)),
    )
    def scatter_kernel(y_hbm, src_hbm, dst_hbm, zeros_hbm, out_hbm,
                       src_v, dst_v, rows_v, acc_sh, sem0, sem1):
        c = lax.axis_index("c")
        s = lax.axis_index("s")
        wid = c * NS + s
        sems = (sem0, sem1)

        zbase = s * ZROWS_PER_TILE
        pltpu.sync_copy(zeros_hbm, acc_sh.at[pl.ds(zbase, ZROWS_PER_TILE)])
        plsc.subcore_barrier()

        for half in range(2):
            pltpu.sync_copy(src_hbm.at[wid, pl.ds(half * HALF, HALF)], src_v)
            pltpu.sync_copy(dst_hbm.at[wid, pl.ds(half * HALF, HALF)], dst_v)

            for b in range(2):
                pltpu.async_copy(y_hbm.at[src_v.at[b]], rows_v.at[b], sems[b])

            def body(jj, _):
                for b in range(2):
                    j = jj * 2 + b
                    pltpu.make_async_copy(
                        y_hbm.at[src_v.at[j]], rows_v.at[b], sems[b]).wait()
                    pltpu.sync_copy(rows_v.at[b], acc_sh.at[dst_v.at[j]],
                                    add=True)

                    @pl.when(j + 2 < HALF)
                    def _():
                        pltpu.async_copy(
                            y_hbm.at[src_v.at[j + 2]], rows_v.at[b], sems[b])
                return 0

            lax.fori_loop(0, HALF // 2, body, 0)

        plsc.subcore_barrier()
        pltpu.sync_copy(acc_sh.at[pl.ds(zbase, ZROWS_PER_TILE)],
                        out_hbm.at[c, pl.ds(zbase, ZROWS_PER_TILE)])

    return scatter_kernel


_make_deg_kernel = functools.cache(_make_deg_kernel)
_make_scatter_kernel = functools.cache(_make_scatter_kernel)


def _tc1_body(x_ref, w1_ref, parts_ref, y1_ref, dinv_ref):
    ones = jnp.ones((NW, 1), jnp.float32)
    deg = lax.dot_general(parts_ref[...], ones,
                          (((0,), (0,)), ((), ())),
                          precision=lax.Precision.HIGHEST,
                          preferred_element_type=jnp.float32)
    deg = lax.slice(deg, (0, 0), (N_NODES, 1)) + 1.0
    dinv = lax.rsqrt(deg)
    xw = jnp.dot(x_ref[...], w1_ref[...],
                 precision=lax.Precision.HIGHEST,
                 preferred_element_type=jnp.float32)
    y1_ref[...] = xw * dinv
    dinv_ref[...] = dinv


def _tc2_body(p_ref, y1_ref, dinv_ref, b1_ref, w2_ref, y2_ref):
    dinv = dinv_ref[...]
    psum = p_ref[0] + p_ref[1]
    srow = lax.slice(psum, (0, 0), (N_NODES, HID_CH)) + y1_ref[...]
    h = jnp.maximum(srow * dinv + b1_ref[...], 0.0)
    y2_ref[...] = jnp.dot(h, w2_ref[...],
                          precision=lax.Precision.HIGHEST,
                          preferred_element_type=jnp.float32) * dinv


def _tc3_body(p_ref, y2_ref, dinv_ref, b2_ref, batch_ref, out_ref):
    psum = p_ref[0] + p_ref[1]
    srow = lax.slice(psum, (0, 0), (N_NODES, OUT_CH)) + y2_ref[...]
    h = srow * dinv_ref[...] + b2_ref[...]
    onehot = (batch_ref[...] == lax.broadcasted_iota(
        jnp.int32, (N_NODES, N_GRAPHS), 1)).astype(jnp.float32)
    seg = lax.dot_general(onehot, h, (((0,), (0,)), ((), ())),
                          precision=lax.Precision.HIGHEST,
                          preferred_element_type=jnp.float32)
    counts = lax.dot_general(onehot, jnp.ones((N_NODES, 1), jnp.float32),
                             (((0,), (0,)), ((), ())),
                             precision=lax.Precision.HIGHEST,
                             preferred_element_type=jnp.float32)
    out_ref[...] = seg / jnp.maximum(counts, 1.0)


_tc1 = pl.pallas_call(
    _tc1_body,
    out_shape=(jax.ShapeDtypeStruct((N_NODES, HID_CH), jnp.float32),
               jax.ShapeDtypeStruct((N_NODES, 1), jnp.float32)))

_tc2 = pl.pallas_call(
    _tc2_body,
    out_shape=jax.ShapeDtypeStruct((N_NODES, OUT_CH), jnp.float32))

_tc3 = pl.pallas_call(
    _tc3_body,
    out_shape=jax.ShapeDtypeStruct((N_GRAPHS, OUT_CH), jnp.float32))


def kernel(x, edge_index, batch, W1, b1, W2, b2):
    src = edge_index[0].astype(jnp.int32)
    dst = edge_index[1].astype(jnp.int32)
    pad = E_PAD - N_EDGES
    src_p = jnp.concatenate([src, jnp.zeros((pad,), jnp.int32)])
    pad_dst = N_NODES + jnp.arange(pad, dtype=jnp.int32) % (ACC_ROWS - N_NODES)
    dst_p = jnp.concatenate([dst, pad_dst])
    src3 = src_p.reshape(NW, NCHUNK, K_EDGE)
    dst3 = dst_p.reshape(NW, NCHUNK, K_EDGE)

    zeros_hid = jnp.zeros((ZROWS_PER_TILE, HID_CH), jnp.float32)
    zeros_out = jnp.zeros((ZROWS_PER_TILE, OUT_CH), jnp.float32)

    deg_kernel = _make_deg_kernel()
    scatter_hid = _make_scatter_kernel(HID_CH)
    scatter_out = _make_scatter_kernel(OUT_CH)

    deg_parts = deg_kernel(dst_p)
    y1, dinv = _tc1(x, W1, deg_parts)
    p1 = scatter_hid(y1, src3, dst3, zeros_hid)
    y2 = _tc2(p1, y1, dinv, b1.reshape(1, HID_CH), W2)
    p2 = scatter_out(y2, src3, dst3, zeros_out)
    return _tc3(p2, y2, dinv, b2.reshape(1, OUT_CH),
                batch.astype(jnp.int32).reshape(N_NODES, 1))
